# baseline (speedup 1.0000x reference)
import functools

import jax
import jax.numpy as jnp
from jax import lax
from jax.experimental import pallas as pl
from jax.experimental.pallas import tpu as pltpu, tpu_sc as plsc

N = 10000
D = 128
E = 320000
NP = 10240
NW = 32
CHUNK = 128
CW = 80
EP = NW * CW * CHUNK
ROWS_PER_TILE = NP // 16
BLK = 1024

_mesh = plsc.VectorSubcoreMesh(core_axis_name="c", subcore_axis_name="s")


def _deg_body(dst_hbm, out_hbm, dst_v, ones_v, acc_sh):
    cid = lax.axis_index("c")
    sid = lax.axis_index("s")
    wid = sid * 2 + cid
    pltpu.sync_copy(dst_hbm.at[wid], dst_v)

    lane = lax.iota(jnp.int32, 16)
    one16 = jnp.where(lane == 0, 1.0, 0.0).astype(jnp.float32)
    zero16 = jnp.zeros((16,), jnp.float32)

    def fill0(i, _):
        for l in range(D // 16):
            ones_v[i, pl.ds(l * 16, 16)] = zero16
        return 0
    lax.fori_loop(0, CHUNK, fill0, 0)

    def zb(b, _):
        pltpu.sync_copy(
            ones_v, acc_sh.at[pl.ds(sid * ROWS_PER_TILE + b * CHUNK, CHUNK)])
        return 0
    lax.fori_loop(0, ROWS_PER_TILE // CHUNK, zb, 0)

    def fill1(i, _):
        ones_v[i, pl.ds(0, 16)] = one16
        return 0
    lax.fori_loop(0, CHUNK, fill1, 0)
    plsc.subcore_barrier()

    def chunk(j, _):
        pltpu.sync_copy(ones_v, acc_sh.at[dst_v.at[j]], add=True)
        return 0
    lax.fori_loop(0, CW, chunk, 0)
    plsc.subcore_barrier()

    r0 = sid * ROWS_PER_TILE
    pltpu.sync_copy(acc_sh.at[pl.ds(r0, ROWS_PER_TILE)],
                    out_hbm.at[cid].at[pl.ds(r0, ROWS_PER_TILE)])


def _agg_body(y_hbm, src_hbm, dst_hbm, out_hbm,
              src_v, dst_v, rows_v, acc_sh, sem):
    cid = lax.axis_index("c")
    sid = lax.axis_index("s")
    wid = sid * 2 + cid
    pltpu.sync_copy(src_hbm.at[wid], src_v)
    pltpu.sync_copy(dst_hbm.at[wid], dst_v)

    zero16 = jnp.zeros((16,), jnp.float32)

    def zr(i, _):
        for l in range(D // 16):
            rows_v[i, pl.ds(l * 16, 16)] = zero16
        return 0
    lax.fori_loop(0, CHUNK, zr, 0)

    def zb(b, _):
        pltpu.sync_copy(
            rows_v, acc_sh.at[pl.ds(sid * ROWS_PER_TILE + b * CHUNK, CHUNK)])
        return 0
    lax.fori_loop(0, ROWS_PER_TILE // CHUNK, zb, 0)
    plsc.subcore_barrier()

    def chunk(j, _):
        pltpu.async_copy(y_hbm.at[src_v.at[j]], rows_v, sem).wait()
        pltpu.sync_copy(rows_v, acc_sh.at[dst_v.at[j]], add=True)
        return 0
    lax.fori_loop(0, CW, chunk, 0)
    plsc.subcore_barrier()

    r0 = sid * ROWS_PER_TILE
    pltpu.sync_copy(acc_sh.at[pl.ds(r0, ROWS_PER_TILE)],
                    out_hbm.at[cid].at[pl.ds(r0, ROWS_PER_TILE)])


_DEG_OUT = jax.ShapeDtypeStruct((2, NP, D), jnp.float32)
_DEG_SCRATCH = [
    pltpu.VMEM((CW, CHUNK), jnp.int32),
    pltpu.VMEM((CHUNK, D), jnp.float32),
    pltpu.VMEM_SHARED((NP, D), jnp.float32),
]
_AGG_OUT = jax.ShapeDtypeStruct((2, NP, D), jnp.float32)
_AGG_SCRATCH = [
    pltpu.VMEM((CW, CHUNK), jnp.int32),
    pltpu.VMEM((CW, CHUNK), jnp.int32),
    pltpu.VMEM((CHUNK, D), jnp.float32),
    pltpu.VMEM_SHARED((NP, D), jnp.float32),
    pltpu.SemaphoreType.DMA,
]

k_deg = pl.kernel(_deg_body, out_type=_DEG_OUT, mesh=_mesh,
                  scratch_types=_DEG_SCRATCH)
k_agg = pl.kernel(_agg_body, out_type=_AGG_OUT, mesh=_mesh,
                  scratch_types=_AGG_SCRATCH)


def _dis_block(degp):
    deg = 1.0 + degp[0, :, 0] + degp[1, :, 0]
    return lax.rsqrt(deg)[:, None]


def _y1_body(x_ref, w_ref, degp_ref, o_ref):
    dis = _dis_block(degp_ref[...])
    o_ref[...] = jnp.dot(x_ref[...], w_ref[...],
                         preferred_element_type=jnp.float32,
                         precision=lax.Precision.HIGHEST) * dis


def _mid_body(p_ref, y_ref, degp_ref, b_ref, w_ref, o_ref):
    dis = _dis_block(degp_ref[...])
    acc = p_ref[0] + p_ref[1] + y_ref[...]
    h = jnp.maximum(acc * dis + b_ref[...], 0.0)
    o_ref[...] = jnp.dot(h, w_ref[...],
                         preferred_element_type=jnp.float32,
                         precision=lax.Precision.HIGHEST) * dis


def _out_body(q_ref, y_ref, degp_ref, b_ref, o_ref):
    dis = _dis_block(degp_ref[...])
    o_ref[...] = (q_ref[0] + q_ref[1] + y_ref[...]) * dis + b_ref[...]


_row_spec = pl.BlockSpec((BLK, D), lambda i: (i, 0))
_pair_spec = pl.BlockSpec((2, BLK, D), lambda i: (0, i, 0))
_deg_spec = pl.BlockSpec((2, BLK, D), lambda i: (0, i, 0))
_w_spec = pl.BlockSpec((D, D), lambda i: (0, 0))
_b_spec = pl.BlockSpec((1, D), lambda i: (0, 0))
_GRID = (NP // BLK,)
_out128 = jax.ShapeDtypeStruct((NP, D), jnp.float32)

_k_y1 = pl.pallas_call(
    _y1_body, grid=_GRID,
    in_specs=[_row_spec, _w_spec, _deg_spec],
    out_specs=_row_spec, out_shape=_out128)

_k_mid = pl.pallas_call(
    _mid_body, grid=_GRID,
    in_specs=[_pair_spec, _row_spec, _deg_spec, _b_spec, _w_spec],
    out_specs=_row_spec, out_shape=_out128)

_k_out = pl.pallas_call(
    _out_body, grid=_GRID,
    in_specs=[_pair_spec, _row_spec, _deg_spec, _b_spec],
    out_specs=_row_spec, out_shape=_out128)


def kernel(x, edge_index, W1, b1, W2, b2):
    ei = edge_index.astype(jnp.int32)
    pad = jnp.full((EP - E,), N, dtype=jnp.int32)
    src = jnp.concatenate([ei[0], pad]).reshape(NW, CW, CHUNK)
    dst = jnp.concatenate([ei[1], pad]).reshape(NW, CW, CHUNK)
    xp = jnp.pad(x, ((0, NP - N), (0, 0)))
    b1r = b1.reshape(1, D)
    b2r = b2.reshape(1, D)

    degp = k_deg(dst)
    y1 = _k_y1(xp, W1, degp)
    p1 = k_agg(y1, src, dst)
    y2 = _k_mid(p1, y1, degp, b1r, W2)
    p2 = k_agg(y2, src, dst)
    z = _k_out(p2, y2, degp, b2r)
    return z[:N]


# speedup vs baseline: 1.5203x; 1.5203x over previous
import functools

import jax
import jax.numpy as jnp
from jax import lax
from jax.experimental import pallas as pl
from jax.experimental.pallas import tpu as pltpu, tpu_sc as plsc

N = 10000
D = 128
E = 320000
NP = 10240
NW = 32
CHUNK = 128
CW = 79
EP = NW * CW * CHUNK
ROWS_PER_TILE = NP // 16
BLK = 1024

_mesh = plsc.VectorSubcoreMesh(core_axis_name="c", subcore_axis_name="s")


def _deg_body(dst_hbm, out_hbm, dst_v, ones_v, acc_sh):
    cid = lax.axis_index("c")
    sid = lax.axis_index("s")
    wid = sid * 2 + cid
    pltpu.sync_copy(dst_hbm.at[wid], dst_v)

    lane = lax.iota(jnp.int32, 16)
    one16 = jnp.where(lane == 0, 1.0, 0.0).astype(jnp.float32)
    zero16 = jnp.zeros((16,), jnp.float32)

    def fill0(i, _):
        for l in range(D // 16):
            ones_v[i, pl.ds(l * 16, 16)] = zero16
        return 0
    lax.fori_loop(0, CHUNK, fill0, 0)

    def zb(b, _):
        pltpu.sync_copy(
            ones_v, acc_sh.at[pl.ds(sid * ROWS_PER_TILE + b * CHUNK, CHUNK)])
        return 0
    lax.fori_loop(0, ROWS_PER_TILE // CHUNK, zb, 0)

    def fill1(i, _):
        ones_v[i, pl.ds(0, 16)] = one16
        return 0
    lax.fori_loop(0, CHUNK, fill1, 0)
    plsc.subcore_barrier()

    def chunk(j, _):
        pltpu.sync_copy(ones_v, acc_sh.at[dst_v.at[j]], add=True)
        return 0
    lax.fori_loop(0, CW, chunk, 0)
    plsc.subcore_barrier()

    r0 = sid * ROWS_PER_TILE
    pltpu.sync_copy(acc_sh.at[pl.ds(r0, ROWS_PER_TILE)],
                    out_hbm.at[cid].at[pl.ds(r0, ROWS_PER_TILE)])


def _agg_body(y_hbm, src_hbm, dst_hbm, out_hbm,
              src_v, dst_v, rows_v, acc_sh, sem):
    cid = lax.axis_index("c")
    sid = lax.axis_index("s")
    wid = sid * 2 + cid
    pltpu.sync_copy(src_hbm.at[wid], src_v)
    pltpu.sync_copy(dst_hbm.at[wid], dst_v)

    zero16 = jnp.zeros((16,), jnp.float32)

    def zr(i, _):
        for l in range(D // 16):
            rows_v[i, pl.ds(l * 16, 16)] = zero16
        return 0
    lax.fori_loop(0, CHUNK, zr, 0)

    def zb(b, _):
        pltpu.sync_copy(
            rows_v, acc_sh.at[pl.ds(sid * ROWS_PER_TILE + b * CHUNK, CHUNK)])
        return 0
    lax.fori_loop(0, ROWS_PER_TILE // CHUNK, zb, 0)
    plsc.subcore_barrier()

    def chunk(j, _):
        pltpu.async_copy(y_hbm.at[src_v.at[j]], rows_v, sem).wait()
        pltpu.sync_copy(rows_v, acc_sh.at[dst_v.at[j]], add=True)
        return 0
    lax.fori_loop(0, CW, chunk, 0)
    plsc.subcore_barrier()

    r0 = sid * ROWS_PER_TILE
    pltpu.sync_copy(acc_sh.at[pl.ds(r0, ROWS_PER_TILE)],
                    out_hbm.at[cid].at[pl.ds(r0, ROWS_PER_TILE)])


_DEG_OUT = jax.ShapeDtypeStruct((2, NP, D), jnp.float32)
_DEG_SCRATCH = [
    pltpu.VMEM((CW, CHUNK), jnp.int32),
    pltpu.VMEM((CHUNK, D), jnp.float32),
    pltpu.VMEM_SHARED((NP, D), jnp.float32),
]
_AGG_OUT = jax.ShapeDtypeStruct((2, NP, D), jnp.float32)
_AGG_SCRATCH = [
    pltpu.VMEM((CW, CHUNK), jnp.int32),
    pltpu.VMEM((CW, CHUNK), jnp.int32),
    pltpu.VMEM((CHUNK, D), jnp.float32),
    pltpu.VMEM_SHARED((NP, D), jnp.float32),
    pltpu.SemaphoreType.DMA,
]

k_deg = pl.kernel(_deg_body, out_type=_DEG_OUT, mesh=_mesh,
                  scratch_types=_DEG_SCRATCH)
k_agg = pl.kernel(_agg_body, out_type=_AGG_OUT, mesh=_mesh,
                  scratch_types=_AGG_SCRATCH)


def _dis_block(degp):
    deg = 1.0 + degp[0, :, 0] + degp[1, :, 0]
    return lax.rsqrt(deg)[:, None]


def _y1_body(x_ref, w_ref, degp_ref, o_ref):
    dis = _dis_block(degp_ref[...])
    o_ref[...] = jnp.dot(x_ref[...], w_ref[...],
                         preferred_element_type=jnp.float32,
                         precision=lax.Precision.HIGHEST) * dis


def _mid_body(p_ref, y_ref, degp_ref, b_ref, w_ref, o_ref):
    dis = _dis_block(degp_ref[...])
    acc = p_ref[0] + p_ref[1] + y_ref[...]
    h = jnp.maximum(acc * dis + b_ref[...], 0.0)
    o_ref[...] = jnp.dot(h, w_ref[...],
                         preferred_element_type=jnp.float32,
                         precision=lax.Precision.HIGHEST) * dis


def _out_body(q_ref, y_ref, degp_ref, b_ref, o_ref):
    dis = _dis_block(degp_ref[...])
    o_ref[...] = (q_ref[0] + q_ref[1] + y_ref[...]) * dis + b_ref[...]


_row_spec = pl.BlockSpec((BLK, D), lambda i: (i, 0))
_pair_spec = pl.BlockSpec((2, BLK, D), lambda i: (0, i, 0))
_deg_spec = pl.BlockSpec((2, BLK, D), lambda i: (0, i, 0))
_w_spec = pl.BlockSpec((D, D), lambda i: (0, 0))
_b_spec = pl.BlockSpec((1, D), lambda i: (0, 0))
_GRID = (NP // BLK,)
_out128 = jax.ShapeDtypeStruct((NP, D), jnp.float32)

_k_y1 = pl.pallas_call(
    _y1_body, grid=_GRID,
    in_specs=[_row_spec, _w_spec, _deg_spec],
    out_specs=_row_spec, out_shape=_out128)

_k_mid = pl.pallas_call(
    _mid_body, grid=_GRID,
    in_specs=[_pair_spec, _row_spec, _deg_spec, _b_spec, _w_spec],
    out_specs=_row_spec, out_shape=_out128)

_k_out = pl.pallas_call(
    _out_body, grid=_GRID,
    in_specs=[_pair_spec, _row_spec, _deg_spec, _b_spec],
    out_specs=_row_spec, out_shape=_out128)


def kernel(x, edge_index, W1, b1, W2, b2):
    ei = edge_index.astype(jnp.int32)
    pad = jnp.full((EP - E,), N, dtype=jnp.int32)
    src = jnp.concatenate([ei[0], pad]).reshape(NW, CW, CHUNK)
    dst = jnp.concatenate([ei[1], pad]).reshape(NW, CW, CHUNK)
    xp = jnp.pad(x, ((0, NP - N), (0, 0)))
    b1r = b1.reshape(1, D)
    b2r = b2.reshape(1, D)

    degp = k_deg(dst)
    y1 = _k_y1(xp, W1, degp)
    p1 = k_agg(y1, src, dst)
    y2 = _k_mid(p1, y1, degp, b1r, W2)
    p2 = k_agg(y2, src, dst)
    z = _k_out(p2, y2, degp, b2r)
    return z[:N]


# speedup vs baseline: 2.6825x; 1.7644x over previous
import functools

import jax
import jax.numpy as jnp
from jax import lax
from jax.experimental import pallas as pl
from jax.experimental.pallas import tpu as pltpu, tpu_sc as plsc

N = 10000
D = 128
E = 320000
NP = 10240
NW = 32
CHUNK = 128
CW = 79
EP = NW * CW * CHUNK
ROWS_PER_TILE = NP // 16
BLK = 1024

_mesh = plsc.VectorSubcoreMesh(core_axis_name="c", subcore_axis_name="s")


def _deg_body(dst_hbm, out_hbm, dst_v, ones_v, acc_sh):
    cid = lax.axis_index("c")
    sid = lax.axis_index("s")
    wid = sid * 2 + cid
    pltpu.sync_copy(dst_hbm.at[wid], dst_v)

    lane = lax.iota(jnp.int32, 16)
    one16 = jnp.where(lane == 0, 1.0, 0.0).astype(jnp.float32)
    zero16 = jnp.zeros((16,), jnp.float32)

    def fill0(i, _):
        for l in range(D // 16):
            ones_v[i, pl.ds(l * 16, 16)] = zero16
        return 0
    lax.fori_loop(0, CHUNK, fill0, 0)

    def zb(b, _):
        pltpu.sync_copy(
            ones_v, acc_sh.at[pl.ds(sid * ROWS_PER_TILE + b * CHUNK, CHUNK)])
        return 0
    lax.fori_loop(0, ROWS_PER_TILE // CHUNK, zb, 0)

    def fill1(i, _):
        ones_v[i, pl.ds(0, 16)] = one16
        return 0
    lax.fori_loop(0, CHUNK, fill1, 0)
    plsc.subcore_barrier()

    def chunk(j, _):
        pltpu.sync_copy(ones_v, acc_sh.at[dst_v.at[j]], add=True)
        return 0
    lax.fori_loop(0, CW, chunk, 0)
    plsc.subcore_barrier()

    r0 = sid * ROWS_PER_TILE
    pltpu.sync_copy(acc_sh.at[pl.ds(r0, ROWS_PER_TILE)],
                    out_hbm.at[cid].at[pl.ds(r0, ROWS_PER_TILE)])


def _agg_body(y_hbm, src_hbm, dst_hbm, out_hbm,
              src_v, dst_v, rows_v, acc_sh, sem):
    cid = lax.axis_index("c")
    sid = lax.axis_index("s")
    wid = sid * 2 + cid
    pltpu.sync_copy(src_hbm.at[wid], src_v)
    pltpu.sync_copy(dst_hbm.at[wid], dst_v)

    zero16 = jnp.zeros((16,), jnp.float32)

    def zr(i, _):
        for l in range(D // 16):
            rows_v[i, pl.ds(l * 16, 16)] = zero16
        return 0
    lax.fori_loop(0, CHUNK, zr, 0)

    def zb(b, _):
        pltpu.sync_copy(
            rows_v, acc_sh.at[pl.ds(sid * ROWS_PER_TILE + b * CHUNK, CHUNK)])
        return 0
    lax.fori_loop(0, ROWS_PER_TILE // CHUNK, zb, 0)
    plsc.subcore_barrier()

    def chunk(j, _):
        pltpu.async_copy(y_hbm.at[src_v.at[j]], rows_v, sem).wait()
        pltpu.sync_copy(rows_v, acc_sh.at[dst_v.at[j]], add=True)
        return 0
    lax.fori_loop(0, CW, chunk, 0)
    plsc.subcore_barrier()

    r0 = sid * ROWS_PER_TILE
    pltpu.sync_copy(acc_sh.at[pl.ds(r0, ROWS_PER_TILE)],
                    out_hbm.at[cid].at[pl.ds(r0, ROWS_PER_TILE)])


_DEG_OUT = jax.ShapeDtypeStruct((2, NP, D), jnp.float32)
_DEG_SCRATCH = [
    pltpu.VMEM((CW, CHUNK), jnp.int32),
    pltpu.VMEM((CHUNK, D), jnp.float32),
    pltpu.VMEM_SHARED((NP, D), jnp.float32),
]
_AGG_OUT = jax.ShapeDtypeStruct((2, NP, D), jnp.float32)
_AGG_SCRATCH = [
    pltpu.VMEM((CW, CHUNK), jnp.int32),
    pltpu.VMEM((CW, CHUNK), jnp.int32),
    pltpu.VMEM((CHUNK, D), jnp.float32),
    pltpu.VMEM_SHARED((NP, D), jnp.float32),
    pltpu.SemaphoreType.DMA,
]

k_deg = pl.kernel(_deg_body, out_type=_DEG_OUT, mesh=_mesh,
                  scratch_types=_DEG_SCRATCH)
k_agg = pl.kernel(_agg_body, out_type=_AGG_OUT, mesh=_mesh,
                  scratch_types=_AGG_SCRATCH)


def _dis_block(degp):
    deg = 1.0 + degp[0, :, 0] + degp[1, :, 0]
    return lax.rsqrt(deg)[:, None]


def _y1_body(x_ref, w_ref, degp_ref, o_ref):
    dis = _dis_block(degp_ref[...])
    o_ref[...] = jnp.dot(x_ref[...], w_ref[...],
                         preferred_element_type=jnp.float32,
                         precision=lax.Precision.HIGHEST) * dis


def _mid_body(p_ref, y_ref, degp_ref, b_ref, w_ref, o_ref):
    dis = _dis_block(degp_ref[...])
    acc = p_ref[0] + p_ref[1] + y_ref[...]
    h = jnp.maximum(acc * dis + b_ref[...], 0.0)
    o_ref[...] = jnp.dot(h, w_ref[...],
                         preferred_element_type=jnp.float32,
                         precision=lax.Precision.HIGHEST) * dis


def _out_body(q_ref, y_ref, degp_ref, b_ref, o_ref):
    dis = _dis_block(degp_ref[...])
    o_ref[...] = (q_ref[0] + q_ref[1] + y_ref[...]) * dis + b_ref[...]


_row_spec = pl.BlockSpec((BLK, D), lambda i: (i, 0))
_pair_spec = pl.BlockSpec((2, BLK, D), lambda i: (0, i, 0))
_deg_spec = pl.BlockSpec((2, BLK, D), lambda i: (0, i, 0))
_w_spec = pl.BlockSpec((D, D), lambda i: (0, 0))
_b_spec = pl.BlockSpec((1, D), lambda i: (0, 0))
_GRID = (NP // BLK,)
_out128 = jax.ShapeDtypeStruct((NP, D), jnp.float32)

_k_y1 = pl.pallas_call(
    _y1_body, grid=_GRID,
    in_specs=[_row_spec, _w_spec, _deg_spec],
    out_specs=_row_spec, out_shape=_out128)

_k_mid = pl.pallas_call(
    _mid_body, grid=_GRID,
    in_specs=[_pair_spec, _row_spec, _deg_spec, _b_spec, _w_spec],
    out_specs=_row_spec, out_shape=_out128)

_k_out = pl.pallas_call(
    _out_body, grid=_GRID,
    in_specs=[_pair_spec, _row_spec, _deg_spec, _b_spec],
    out_specs=_row_spec, out_shape=_out128)


def kernel(x, edge_index, W1, b1, W2, b2):
    ei = edge_index.astype(jnp.int32)
    pad = N + 1 + (jnp.arange(EP - E, dtype=jnp.int32) % (NP - N - 1))
    src = jnp.concatenate([ei[0], pad]).reshape(NW, CW, CHUNK)
    dst = jnp.concatenate([ei[1], pad]).reshape(NW, CW, CHUNK)
    xp = jnp.pad(x, ((0, NP - N), (0, 0)))
    b1r = b1.reshape(1, D)
    b2r = b2.reshape(1, D)

    degp = k_deg(dst)
    y1 = _k_y1(xp, W1, degp)
    p1 = k_agg(y1, src, dst)
    y2 = _k_mid(p1, y1, degp, b1r, W2)
    p2 = k_agg(y2, src, dst)
    z = _k_out(p2, y2, degp, b2r)
    return z[:N]


# speedup vs baseline: 3.4013x; 1.2680x over previous
import functools

import jax
import jax.numpy as jnp
from jax import lax
from jax.experimental import pallas as pl
from jax.experimental.pallas import tpu as pltpu, tpu_sc as plsc

N = 10000
D = 128
E = 320000
NP = 10240
NW = 32
CHUNK = 128
CW = 80
EP = NW * CW * CHUNK
ROWS_PER_TILE = NP // 16
BLK = 1024

_mesh = plsc.VectorSubcoreMesh(core_axis_name="c", subcore_axis_name="s")


def _deg_body(dst_hbm, out_hbm, dst_v, ones_v, acc_sh):
    cid = lax.axis_index("c")
    sid = lax.axis_index("s")
    wid = sid * 2 + cid
    pltpu.sync_copy(dst_hbm.at[wid], dst_v)

    lane = lax.iota(jnp.int32, 16)
    one16 = jnp.where(lane == 0, 1.0, 0.0).astype(jnp.float32)
    zero16 = jnp.zeros((16,), jnp.float32)

    def fill0(i, _):
        for l in range(D // 16):
            ones_v[i, pl.ds(l * 16, 16)] = zero16
        return 0
    lax.fori_loop(0, CHUNK, fill0, 0)

    def zb(b, _):
        pltpu.sync_copy(
            ones_v, acc_sh.at[pl.ds(sid * ROWS_PER_TILE + b * CHUNK, CHUNK)])
        return 0
    lax.fori_loop(0, ROWS_PER_TILE // CHUNK, zb, 0)

    def fill1(i, _):
        ones_v[i, pl.ds(0, 16)] = one16
        return 0
    lax.fori_loop(0, CHUNK, fill1, 0)
    plsc.subcore_barrier()

    def chunk(j, _):
        pltpu.sync_copy(ones_v, acc_sh.at[dst_v.at[j]], add=True)
        return 0
    lax.fori_loop(0, CW, chunk, 0)
    plsc.subcore_barrier()

    r0 = sid * ROWS_PER_TILE
    pltpu.sync_copy(acc_sh.at[pl.ds(r0, ROWS_PER_TILE)],
                    out_hbm.at[cid].at[pl.ds(r0, ROWS_PER_TILE)])


def _agg_body(y_hbm, ei_hbm, out_hbm,
              ib0, ib1, rows_a, rows_b, acc_sh, sem_i, sem_g):
    cid = lax.axis_index("c")
    sid = lax.axis_index("s")
    wid = sid * 2 + cid
    base = wid * CW

    zero16 = jnp.zeros((16,), jnp.float32)

    def zr(i, _):
        for l in range(D // 16):
            rows_a[i, pl.ds(l * 16, 16)] = zero16
        return 0
    lax.fori_loop(0, CHUNK, zr, 0)

    def zb(b, _):
        pltpu.sync_copy(
            rows_a,
            acc_sh.at[pl.ds(sid * ROWS_PER_TILE + b * CHUNK, CHUNK)])
        return 0
    lax.fori_loop(0, ROWS_PER_TILE // CHUNK, zb, 0)
    plsc.subcore_barrier()

    ib = (ib0, ib1)
    rows = (rows_a, rows_b)

    pltpu.async_copy(ei_hbm.at[base + 0], ib0, sem_i.at[0])
    pltpu.async_copy(ei_hbm.at[base + 1], ib1, sem_i.at[1])
    pltpu.make_async_copy(ei_hbm.at[base], ib0, sem_i.at[0]).wait()
    pltpu.async_copy(y_hbm.at[ib0.at[0]], rows_a, sem_g.at[0])

    def chunk2(jj, _):
        for b in range(2):
            j = 2 * jj + b
            nb = 1 - b

            @pl.when(j + 1 < CW)
            def _():
                pltpu.make_async_copy(ei_hbm.at[base + j + 1], ib[nb],
                                      sem_i.at[nb]).wait()
                pltpu.async_copy(y_hbm.at[ib[nb].at[0]], rows[nb],
                                 sem_g.at[nb])

            pltpu.make_async_copy(y_hbm.at[ib[b].at[0]], rows[b],
                                  sem_g.at[b]).wait()
            pltpu.sync_copy(rows[b], acc_sh.at[ib[b].at[1]], add=True)

            @pl.when(j + 2 < CW)
            def _():
                pltpu.async_copy(ei_hbm.at[base + j + 2], ib[b],
                                 sem_i.at[b])
        return 0
    lax.fori_loop(0, CW // 2, chunk2, 0)
    plsc.subcore_barrier()

    r0 = sid * ROWS_PER_TILE
    pltpu.sync_copy(acc_sh.at[pl.ds(r0, ROWS_PER_TILE)],
                    out_hbm.at[cid].at[pl.ds(r0, ROWS_PER_TILE)])


_DEG_OUT = jax.ShapeDtypeStruct((2, NP, D), jnp.float32)
_DEG_SCRATCH = [
    pltpu.VMEM((CW, CHUNK), jnp.int32),
    pltpu.VMEM((CHUNK, D), jnp.float32),
    pltpu.VMEM_SHARED((NP, D), jnp.float32),
]
_AGG_OUT = jax.ShapeDtypeStruct((2, NP, D), jnp.float32)
_AGG_SCRATCH = [
    pltpu.VMEM((2, CHUNK), jnp.int32),
    pltpu.VMEM((2, CHUNK), jnp.int32),
    pltpu.VMEM((CHUNK, D), jnp.float32),
    pltpu.VMEM((CHUNK, D), jnp.float32),
    pltpu.VMEM_SHARED((NP, D), jnp.float32),
    pltpu.SemaphoreType.DMA((2,)),
    pltpu.SemaphoreType.DMA((2,)),
]

k_deg = pl.kernel(_deg_body, out_type=_DEG_OUT, mesh=_mesh,
                  scratch_types=_DEG_SCRATCH)
k_agg = pl.kernel(_agg_body, out_type=_AGG_OUT, mesh=_mesh,
                  scratch_types=_AGG_SCRATCH)


def _dis_block(degp):
    deg = 1.0 + degp[0, :, 0] + degp[1, :, 0]
    return lax.rsqrt(deg)[:, None]


def _y1_body(x_ref, w_ref, degp_ref, o_ref):
    dis = _dis_block(degp_ref[...])
    o_ref[...] = jnp.dot(x_ref[...], w_ref[...],
                         preferred_element_type=jnp.float32,
                         precision=lax.Precision.HIGHEST) * dis


def _mid_body(p_ref, y_ref, degp_ref, b_ref, w_ref, o_ref):
    dis = _dis_block(degp_ref[...])
    acc = p_ref[0] + p_ref[1] + y_ref[...]
    h = jnp.maximum(acc * dis + b_ref[...], 0.0)
    o_ref[...] = jnp.dot(h, w_ref[...],
                         preferred_element_type=jnp.float32,
                         precision=lax.Precision.HIGHEST) * dis


def _out_body(q_ref, y_ref, degp_ref, b_ref, o_ref):
    dis = _dis_block(degp_ref[...])
    o_ref[...] = (q_ref[0] + q_ref[1] + y_ref[...]) * dis + b_ref[...]


_row_spec = pl.BlockSpec((BLK, D), lambda i: (i, 0))
_pair_spec = pl.BlockSpec((2, BLK, D), lambda i: (0, i, 0))
_deg_spec = pl.BlockSpec((2, BLK, D), lambda i: (0, i, 0))
_w_spec = pl.BlockSpec((D, D), lambda i: (0, 0))
_b_spec = pl.BlockSpec((1, D), lambda i: (0, 0))
_GRID = (NP // BLK,)
_out128 = jax.ShapeDtypeStruct((NP, D), jnp.float32)

_k_y1 = pl.pallas_call(
    _y1_body, grid=_GRID,
    in_specs=[_row_spec, _w_spec, _deg_spec],
    out_specs=_row_spec, out_shape=_out128)

_k_mid = pl.pallas_call(
    _mid_body, grid=_GRID,
    in_specs=[_pair_spec, _row_spec, _deg_spec, _b_spec, _w_spec],
    out_specs=_row_spec, out_shape=_out128)

_k_out = pl.pallas_call(
    _out_body, grid=_GRID,
    in_specs=[_pair_spec, _row_spec, _deg_spec, _b_spec],
    out_specs=_row_spec, out_shape=_out128)


def kernel(x, edge_index, W1, b1, W2, b2):
    ei = edge_index.astype(jnp.int32)
    pad = N + 1 + (jnp.arange(EP - E, dtype=jnp.int32) % (NP - N - 1))
    src = jnp.concatenate([ei[0], pad]).reshape(NW, CW, CHUNK)
    dst = jnp.concatenate([ei[1], pad]).reshape(NW, CW, CHUNK)
    eic = jnp.stack([src, dst], axis=2).reshape(NW * CW, 2, CHUNK)
    xp = jnp.pad(x, ((0, NP - N), (0, 0)))
    b1r = b1.reshape(1, D)
    b2r = b2.reshape(1, D)

    degp = k_deg(dst)
    y1 = _k_y1(xp, W1, degp)
    p1 = k_agg(y1, eic)
    y2 = _k_mid(p1, y1, degp, b1r, W2)
    p2 = k_agg(y2, eic)
    z = _k_out(p2, y2, degp, b2r)
    return z[:N]


# speedup vs baseline: 3.4308x; 1.0087x over previous
import functools

import jax
import jax.numpy as jnp
from jax import lax
from jax.experimental import pallas as pl
from jax.experimental.pallas import tpu as pltpu, tpu_sc as plsc

N = 10000
D = 128
E = 320000
NP = 10240
NW = 32
CHUNK = 128
CW = 80
EP = NW * CW * CHUNK
ROWS_PER_TILE = NP // 16
BLK = 1024

_mesh = plsc.VectorSubcoreMesh(core_axis_name="c", subcore_axis_name="s")


def _deg_body(dst_hbm, out_hbm, dst_v, ones_v, acc_sh):
    cid = lax.axis_index("c")
    sid = lax.axis_index("s")
    wid = sid * 2 + cid
    pltpu.sync_copy(dst_hbm.at[wid], dst_v)

    lane = lax.iota(jnp.int32, 16)
    one16 = jnp.where(lane == 0, 1.0, 0.0).astype(jnp.float32)
    zero16 = jnp.zeros((16,), jnp.float32)

    def fill0(i, _):
        for l in range(D // 16):
            ones_v[i, pl.ds(l * 16, 16)] = zero16
        return 0
    lax.fori_loop(0, CHUNK, fill0, 0)

    def zb(b, _):
        pltpu.sync_copy(
            ones_v, acc_sh.at[pl.ds(sid * ROWS_PER_TILE + b * CHUNK, CHUNK)])
        return 0
    lax.fori_loop(0, ROWS_PER_TILE // CHUNK, zb, 0)

    def fill1(i, _):
        ones_v[i, pl.ds(0, 16)] = one16
        return 0
    lax.fori_loop(0, CHUNK, fill1, 0)
    plsc.subcore_barrier()

    def chunk(j, _):
        pltpu.sync_copy(ones_v, acc_sh.at[dst_v.at[j]], add=True)
        return 0
    lax.fori_loop(0, CW, chunk, 0)
    plsc.subcore_barrier()

    r0 = sid * ROWS_PER_TILE
    pltpu.sync_copy(acc_sh.at[pl.ds(r0, ROWS_PER_TILE)],
                    out_hbm.at[cid].at[pl.ds(r0, ROWS_PER_TILE)])


def _agg_body(y_hbm, ei_hbm, out_hbm,
              ib0, ib1, rows_a, rows_b, acc_sh, sem_i, sem_g):
    cid = lax.axis_index("c")
    sid = lax.axis_index("s")
    wid = sid * 2 + cid
    base = wid * CW

    zero16 = jnp.zeros((16,), jnp.float32)

    def zr(i, _):
        for l in range(D // 16):
            rows_a[i, pl.ds(l * 16, 16)] = zero16
        return 0
    lax.fori_loop(0, CHUNK, zr, 0)

    def zb(b, _):
        pltpu.sync_copy(
            rows_a,
            acc_sh.at[pl.ds(sid * ROWS_PER_TILE + b * CHUNK, CHUNK)])
        return 0
    lax.fori_loop(0, ROWS_PER_TILE // CHUNK, zb, 0)
    plsc.subcore_barrier()

    ib = (ib0, ib1)
    rows = (rows_a, rows_b)

    pltpu.async_copy(ei_hbm.at[base + 0], ib0, sem_i.at[0])
    pltpu.async_copy(ei_hbm.at[base + 1], ib1, sem_i.at[1])
    pltpu.make_async_copy(ei_hbm.at[base], ib0, sem_i.at[0]).wait()
    pltpu.async_copy(y_hbm.at[ib0.at[0]], rows_a, sem_g.at[0])

    def chunk2(jj, _):
        for b in range(2):
            j = 2 * jj + b
            nb = 1 - b

            @pl.when(j + 1 < CW)
            def _():
                pltpu.make_async_copy(ei_hbm.at[base + j + 1], ib[nb],
                                      sem_i.at[nb]).wait()
                pltpu.async_copy(y_hbm.at[ib[nb].at[0]], rows[nb],
                                 sem_g.at[nb])

            pltpu.make_async_copy(y_hbm.at[ib[b].at[0]], rows[b],
                                  sem_g.at[b]).wait()
            pltpu.sync_copy(rows[b], acc_sh.at[ib[b].at[1]], add=True)

            @pl.when(j + 2 < CW)
            def _():
                pltpu.async_copy(ei_hbm.at[base + j + 2], ib[b],
                                 sem_i.at[b])
        return 0
    lax.fori_loop(0, CW // 2, chunk2, 0)
    plsc.subcore_barrier()

    r0 = sid * ROWS_PER_TILE
    pltpu.sync_copy(acc_sh.at[pl.ds(r0, ROWS_PER_TILE)],
                    out_hbm.at[cid].at[pl.ds(r0, ROWS_PER_TILE)])


_DEG_OUT = jax.ShapeDtypeStruct((2, NP, D), jnp.float32)
_DEG_SCRATCH = [
    pltpu.VMEM((CW, CHUNK), jnp.int32),
    pltpu.VMEM((CHUNK, D), jnp.float32),
    pltpu.VMEM_SHARED((NP, D), jnp.float32),
]
_AGG_OUT = jax.ShapeDtypeStruct((2, NP, D), jnp.float32)
_AGG_SCRATCH = [
    pltpu.VMEM((2, CHUNK), jnp.int32),
    pltpu.VMEM((2, CHUNK), jnp.int32),
    pltpu.VMEM((CHUNK, D), jnp.float32),
    pltpu.VMEM((CHUNK, D), jnp.float32),
    pltpu.VMEM_SHARED((NP, D), jnp.float32),
    pltpu.SemaphoreType.DMA((2,)),
    pltpu.SemaphoreType.DMA((2,)),
]

k_deg = pl.kernel(_deg_body, out_type=_DEG_OUT, mesh=_mesh,
                  scratch_types=_DEG_SCRATCH)
k_agg = pl.kernel(_agg_body, out_type=_AGG_OUT, mesh=_mesh,
                  scratch_types=_AGG_SCRATCH)


def _dis_block(degp):
    deg = 1.0 + degp[0, :, 0] + degp[1, :, 0]
    return lax.rsqrt(deg)[:, None]


def _xw_body(x_ref, w_ref, o_ref):
    o_ref[...] = jnp.dot(x_ref[...], w_ref[...],
                         preferred_element_type=jnp.float32,
                         precision=lax.Precision.HIGHEST)


def _scale_body(x_ref, degp_ref, o_ref):
    o_ref[...] = x_ref[...] * _dis_block(degp_ref[...])


def _mid_body(p_ref, y_ref, degp_ref, b_ref, w_ref, o_ref):
    dis = _dis_block(degp_ref[...])
    acc = p_ref[0] + p_ref[1] + y_ref[...]
    h = jnp.maximum(acc * dis + b_ref[...], 0.0)
    o_ref[...] = jnp.dot(h, w_ref[...],
                         preferred_element_type=jnp.float32,
                         precision=lax.Precision.HIGHEST) * dis


def _out_body(q_ref, y_ref, degp_ref, b_ref, o_ref):
    dis = _dis_block(degp_ref[...])
    o_ref[...] = (q_ref[0] + q_ref[1] + y_ref[...]) * dis + b_ref[...]


_row_spec = pl.BlockSpec((BLK, D), lambda i: (i, 0))
_pair_spec = pl.BlockSpec((2, BLK, D), lambda i: (0, i, 0))
_deg_spec = pl.BlockSpec((2, BLK, D), lambda i: (0, i, 0))
_w_spec = pl.BlockSpec((D, D), lambda i: (0, 0))
_b_spec = pl.BlockSpec((1, D), lambda i: (0, 0))
_GRID = (NP // BLK,)
_out128 = jax.ShapeDtypeStruct((NP, D), jnp.float32)

_k_xw = pl.pallas_call(
    _xw_body, grid=_GRID,
    in_specs=[_row_spec, _w_spec],
    out_specs=_row_spec, out_shape=_out128)

_k_scale = pl.pallas_call(
    _scale_body, grid=_GRID,
    in_specs=[_row_spec, _deg_spec],
    out_specs=_row_spec, out_shape=_out128)

_k_mid = pl.pallas_call(
    _mid_body, grid=_GRID,
    in_specs=[_pair_spec, _row_spec, _deg_spec, _b_spec, _w_spec],
    out_specs=_row_spec, out_shape=_out128)

_k_out = pl.pallas_call(
    _out_body, grid=_GRID,
    in_specs=[_pair_spec, _row_spec, _deg_spec, _b_spec],
    out_specs=_row_spec, out_shape=_out128)


def kernel(x, edge_index, W1, b1, W2, b2):
    ei = edge_index.astype(jnp.int32)
    pad = N + 1 + (jnp.arange(EP - E, dtype=jnp.int32) % (NP - N - 1))
    src = jnp.concatenate([ei[0], pad]).reshape(NW, CW, CHUNK)
    dst = jnp.concatenate([ei[1], pad]).reshape(NW, CW, CHUNK)
    eic = jnp.stack([src, dst], axis=2).reshape(NW * CW, 2, CHUNK)
    xp = jnp.pad(x, ((0, NP - N), (0, 0)))
    b1r = b1.reshape(1, D)
    b2r = b2.reshape(1, D)

    xw = _k_xw(xp, W1)
    degp = k_deg(dst)
    y1 = _k_scale(xw, degp)
    p1 = k_agg(y1, eic)
    y2 = _k_mid(p1, y1, degp, b1r, W2)
    p2 = k_agg(y2, eic)
    z = _k_out(p2, y2, degp, b2r)
    return z[:N]


# speedup vs baseline: 3.4390x; 1.0024x over previous
import functools

import jax
import jax.numpy as jnp
from jax import lax
from jax.experimental import pallas as pl
from jax.experimental.pallas import tpu as pltpu, tpu_sc as plsc

N = 10000
D = 128
E = 320000
NP = 10240
NW = 32
CHUNK = 128
CW = 80
EP = NW * CW * CHUNK
ROWS_PER_TILE = NP // 16
BLK = 1024

_mesh = plsc.VectorSubcoreMesh(core_axis_name="c", subcore_axis_name="s")


def _deg_body(dst_hbm, out_hbm, dst_v, ones_v, acc_sh, sem):
    cid = lax.axis_index("c")
    sid = lax.axis_index("s")
    wid = sid * 2 + cid
    pltpu.sync_copy(dst_hbm.at[wid], dst_v)

    lane = lax.iota(jnp.int32, 16)
    one16 = jnp.where(lane == 0, 1.0, 0.0).astype(jnp.float32)
    zero16 = jnp.zeros((16,), jnp.float32)

    def fill0(i, _):
        for l in range(D // 16):
            ones_v[i, pl.ds(l * 16, 16)] = zero16
        return 0
    lax.fori_loop(0, CHUNK, fill0, 0)

    def zb(b, _):
        pltpu.sync_copy(
            ones_v, acc_sh.at[pl.ds(sid * ROWS_PER_TILE + b * CHUNK, CHUNK)])
        return 0
    lax.fori_loop(0, ROWS_PER_TILE // CHUNK, zb, 0)

    def fill1(i, _):
        ones_v[i, pl.ds(0, 16)] = one16
        return 0
    lax.fori_loop(0, CHUNK, fill1, 0)
    plsc.subcore_barrier()

    pltpu.async_copy(ones_v, acc_sh.at[dst_v.at[0]], sem, add=True)

    def chunk(j, _):
        @pl.when(j + 1 < CW)
        def _():
            pltpu.async_copy(ones_v, acc_sh.at[dst_v.at[j + 1]], sem,
                             add=True)
        pltpu.make_async_copy(ones_v, acc_sh.at[dst_v.at[j]], sem).wait()
        return 0
    lax.fori_loop(0, CW, chunk, 0)
    plsc.subcore_barrier()

    r0 = sid * ROWS_PER_TILE
    pltpu.sync_copy(acc_sh.at[pl.ds(r0, ROWS_PER_TILE)],
                    out_hbm.at[cid].at[pl.ds(r0, ROWS_PER_TILE)])


def _agg_body(y_hbm, ei_hbm, out_hbm,
              ib0, ib1, rows_a, rows_b, acc_sh, sem_i, sem_g):
    cid = lax.axis_index("c")
    sid = lax.axis_index("s")
    wid = sid * 2 + cid
    base = wid * CW

    zero16 = jnp.zeros((16,), jnp.float32)

    def zr(i, _):
        for l in range(D // 16):
            rows_a[i, pl.ds(l * 16, 16)] = zero16
        return 0
    lax.fori_loop(0, CHUNK, zr, 0)

    def zb(b, _):
        pltpu.sync_copy(
            rows_a,
            acc_sh.at[pl.ds(sid * ROWS_PER_TILE + b * CHUNK, CHUNK)])
        return 0
    lax.fori_loop(0, ROWS_PER_TILE // CHUNK, zb, 0)
    plsc.subcore_barrier()

    ib = (ib0, ib1)
    rows = (rows_a, rows_b)

    pltpu.async_copy(ei_hbm.at[base + 0], ib0, sem_i.at[0])
    pltpu.async_copy(ei_hbm.at[base + 1], ib1, sem_i.at[1])
    pltpu.make_async_copy(ei_hbm.at[base], ib0, sem_i.at[0]).wait()
    pltpu.async_copy(y_hbm.at[ib0.at[0]], rows_a, sem_g.at[0])

    def chunk2(jj, _):
        for b in range(2):
            j = 2 * jj + b
            nb = 1 - b

            @pl.when(j + 1 < CW)
            def _():
                pltpu.make_async_copy(ei_hbm.at[base + j + 1], ib[nb],
                                      sem_i.at[nb]).wait()
                pltpu.async_copy(y_hbm.at[ib[nb].at[0]], rows[nb],
                                 sem_g.at[nb])

            pltpu.make_async_copy(y_hbm.at[ib[b].at[0]], rows[b],
                                  sem_g.at[b]).wait()
            pltpu.sync_copy(rows[b], acc_sh.at[ib[b].at[1]], add=True)

            @pl.when(j + 2 < CW)
            def _():
                pltpu.async_copy(ei_hbm.at[base + j + 2], ib[b],
                                 sem_i.at[b])
        return 0
    lax.fori_loop(0, CW // 2, chunk2, 0)
    plsc.subcore_barrier()

    r0 = sid * ROWS_PER_TILE
    pltpu.sync_copy(acc_sh.at[pl.ds(r0, ROWS_PER_TILE)],
                    out_hbm.at[cid].at[pl.ds(r0, ROWS_PER_TILE)])


_DEG_OUT = jax.ShapeDtypeStruct((2, NP, D), jnp.float32)
_DEG_SCRATCH = [
    pltpu.VMEM((CW, CHUNK), jnp.int32),
    pltpu.VMEM((CHUNK, D), jnp.float32),
    pltpu.VMEM_SHARED((NP, D), jnp.float32),
    pltpu.SemaphoreType.DMA,
]
_AGG_OUT = jax.ShapeDtypeStruct((2, NP, D), jnp.float32)
_AGG_SCRATCH = [
    pltpu.VMEM((2, CHUNK), jnp.int32),
    pltpu.VMEM((2, CHUNK), jnp.int32),
    pltpu.VMEM((CHUNK, D), jnp.float32),
    pltpu.VMEM((CHUNK, D), jnp.float32),
    pltpu.VMEM_SHARED((NP, D), jnp.float32),
    pltpu.SemaphoreType.DMA((2,)),
    pltpu.SemaphoreType.DMA((2,)),
]

k_deg = pl.kernel(_deg_body, out_type=_DEG_OUT, mesh=_mesh,
                  scratch_types=_DEG_SCRATCH)
k_agg = pl.kernel(_agg_body, out_type=_AGG_OUT, mesh=_mesh,
                  scratch_types=_AGG_SCRATCH)


def _dis_block(degp):
    deg = 1.0 + degp[0, :, 0] + degp[1, :, 0]
    return lax.rsqrt(deg)[:, None]


def _xw_body(x_ref, w_ref, o_ref):
    o_ref[...] = jnp.dot(x_ref[...], w_ref[...],
                         preferred_element_type=jnp.float32,
                         precision=lax.Precision.HIGHEST)


def _scale_body(x_ref, degp_ref, o_ref):
    o_ref[...] = x_ref[...] * _dis_block(degp_ref[...])


def _mid_body(p_ref, y_ref, degp_ref, b_ref, w_ref, o_ref):
    dis = _dis_block(degp_ref[...])
    acc = p_ref[0] + p_ref[1] + y_ref[...]
    h = jnp.maximum(acc * dis + b_ref[...], 0.0)
    o_ref[...] = jnp.dot(h, w_ref[...],
                         preferred_element_type=jnp.float32,
                         precision=lax.Precision.HIGHEST) * dis


def _out_body(q_ref, y_ref, degp_ref, b_ref, o_ref):
    dis = _dis_block(degp_ref[...])
    o_ref[...] = (q_ref[0] + q_ref[1] + y_ref[...]) * dis + b_ref[...]


_row_spec = pl.BlockSpec((BLK, D), lambda i: (i, 0))
_pair_spec = pl.BlockSpec((2, BLK, D), lambda i: (0, i, 0))
_deg_spec = pl.BlockSpec((2, BLK, D), lambda i: (0, i, 0))
_w_spec = pl.BlockSpec((D, D), lambda i: (0, 0))
_b_spec = pl.BlockSpec((1, D), lambda i: (0, 0))
_GRID = (NP // BLK,)
_out128 = jax.ShapeDtypeStruct((NP, D), jnp.float32)

_k_xw = pl.pallas_call(
    _xw_body, grid=_GRID,
    in_specs=[_row_spec, _w_spec],
    out_specs=_row_spec, out_shape=_out128)

_k_scale = pl.pallas_call(
    _scale_body, grid=_GRID,
    in_specs=[_row_spec, _deg_spec],
    out_specs=_row_spec, out_shape=_out128)

_k_mid = pl.pallas_call(
    _mid_body, grid=_GRID,
    in_specs=[_pair_spec, _row_spec, _deg_spec, _b_spec, _w_spec],
    out_specs=_row_spec, out_shape=_out128)

_k_out = pl.pallas_call(
    _out_body, grid=_GRID,
    in_specs=[_pair_spec, _row_spec, _deg_spec, _b_spec],
    out_specs=_row_spec, out_shape=_out128)


def kernel(x, edge_index, W1, b1, W2, b2):
    ei = edge_index.astype(jnp.int32)
    pad = N + 1 + (jnp.arange(EP - E, dtype=jnp.int32) % (NP - N - 1))
    src = jnp.concatenate([ei[0], pad]).reshape(NW, CW, CHUNK)
    dst = jnp.concatenate([ei[1], pad]).reshape(NW, CW, CHUNK)
    eic = jnp.stack([src, dst], axis=2).reshape(NW * CW, 2, CHUNK)
    xp = jnp.pad(x, ((0, NP - N), (0, 0)))
    b1r = b1.reshape(1, D)
    b2r = b2.reshape(1, D)

    xw = _k_xw(xp, W1)
    degp = k_deg(dst)
    y1 = _k_scale(xw, degp)
    p1 = k_agg(y1, eic)
    y2 = _k_mid(p1, y1, degp, b1r, W2)
    p2 = k_agg(y2, eic)
    z = _k_out(p2, y2, degp, b2r)
    return z[:N]


# speedup vs baseline: 3.4920x; 1.0154x over previous
import functools

import jax
import jax.numpy as jnp
from jax import lax
from jax.experimental import pallas as pl
from jax.experimental.pallas import tpu as pltpu, tpu_sc as plsc

N = 10000
D = 128
E = 320000
NP = 10240
NW = 32
CHUNK = 128
CW = 80
EP = NW * CW * CHUNK
ROWS_PER_TILE = NP // 16
BLK = 2048

_mesh = plsc.VectorSubcoreMesh(core_axis_name="c", subcore_axis_name="s")


def _deg_body(dst_hbm, out_hbm, dst_v, ones_v, acc_sh, sem):
    cid = lax.axis_index("c")
    sid = lax.axis_index("s")
    wid = sid * 2 + cid
    pltpu.sync_copy(dst_hbm.at[wid], dst_v)

    lane = lax.iota(jnp.int32, 16)
    one16 = jnp.where(lane == 0, 1.0, 0.0).astype(jnp.float32)
    zero16 = jnp.zeros((16,), jnp.float32)

    def fill0(i, _):
        for l in range(D // 16):
            ones_v[i, pl.ds(l * 16, 16)] = zero16
        return 0
    lax.fori_loop(0, CHUNK, fill0, 0)

    def zb(b, _):
        pltpu.sync_copy(
            ones_v, acc_sh.at[pl.ds(sid * ROWS_PER_TILE + b * CHUNK, CHUNK)])
        return 0
    lax.fori_loop(0, ROWS_PER_TILE // CHUNK, zb, 0)

    def fill1(i, _):
        ones_v[i, pl.ds(0, 16)] = one16
        return 0
    lax.fori_loop(0, CHUNK, fill1, 0)
    plsc.subcore_barrier()

    pltpu.async_copy(ones_v, acc_sh.at[dst_v.at[0]], sem, add=True)

    def chunk(j, _):
        @pl.when(j + 1 < CW)
        def _():
            pltpu.async_copy(ones_v, acc_sh.at[dst_v.at[j + 1]], sem,
                             add=True)
        pltpu.make_async_copy(ones_v, acc_sh.at[dst_v.at[j]], sem).wait()
        return 0
    lax.fori_loop(0, CW, chunk, 0)
    plsc.subcore_barrier()

    r0 = sid * ROWS_PER_TILE
    pltpu.sync_copy(acc_sh.at[pl.ds(r0, ROWS_PER_TILE)],
                    out_hbm.at[cid].at[pl.ds(r0, ROWS_PER_TILE)])


def _agg_body(y_hbm, ei_hbm, out_hbm,
              ib0, ib1, rows_a, rows_b, acc_sh, sem_i, sem_g):
    cid = lax.axis_index("c")
    sid = lax.axis_index("s")
    wid = sid * 2 + cid
    base = wid * CW

    zero16 = jnp.zeros((16,), jnp.float32)

    def zr(i, _):
        for l in range(D // 16):
            rows_a[i, pl.ds(l * 16, 16)] = zero16
        return 0
    lax.fori_loop(0, CHUNK, zr, 0)

    def zb(b, _):
        pltpu.sync_copy(
            rows_a,
            acc_sh.at[pl.ds(sid * ROWS_PER_TILE + b * CHUNK, CHUNK)])
        return 0
    lax.fori_loop(0, ROWS_PER_TILE // CHUNK, zb, 0)
    plsc.subcore_barrier()

    ib = (ib0, ib1)
    rows = (rows_a, rows_b)

    pltpu.async_copy(ei_hbm.at[base + 0], ib0, sem_i.at[0])
    pltpu.async_copy(ei_hbm.at[base + 1], ib1, sem_i.at[1])
    pltpu.make_async_copy(ei_hbm.at[base], ib0, sem_i.at[0]).wait()
    pltpu.async_copy(y_hbm.at[ib0.at[0]], rows_a, sem_g.at[0])

    def chunk2(jj, _):
        for b in range(2):
            j = 2 * jj + b
            nb = 1 - b

            @pl.when(j + 1 < CW)
            def _():
                pltpu.make_async_copy(ei_hbm.at[base + j + 1], ib[nb],
                                      sem_i.at[nb]).wait()
                pltpu.async_copy(y_hbm.at[ib[nb].at[0]], rows[nb],
                                 sem_g.at[nb])

            pltpu.make_async_copy(y_hbm.at[ib[b].at[0]], rows[b],
                                  sem_g.at[b]).wait()
            pltpu.sync_copy(rows[b], acc_sh.at[ib[b].at[1]], add=True)

            @pl.when(j + 2 < CW)
            def _():
                pltpu.async_copy(ei_hbm.at[base + j + 2], ib[b],
                                 sem_i.at[b])
        return 0
    lax.fori_loop(0, CW // 2, chunk2, 0)
    plsc.subcore_barrier()

    r0 = sid * ROWS_PER_TILE
    pltpu.sync_copy(acc_sh.at[pl.ds(r0, ROWS_PER_TILE)],
                    out_hbm.at[cid].at[pl.ds(r0, ROWS_PER_TILE)])


_DEG_OUT = jax.ShapeDtypeStruct((2, NP, D), jnp.float32)
_DEG_SCRATCH = [
    pltpu.VMEM((CW, CHUNK), jnp.int32),
    pltpu.VMEM((CHUNK, D), jnp.float32),
    pltpu.VMEM_SHARED((NP, D), jnp.float32),
    pltpu.SemaphoreType.DMA,
]
_AGG_OUT = jax.ShapeDtypeStruct((2, NP, D), jnp.float32)
_AGG_SCRATCH = [
    pltpu.VMEM((2, CHUNK), jnp.int32),
    pltpu.VMEM((2, CHUNK), jnp.int32),
    pltpu.VMEM((CHUNK, D), jnp.float32),
    pltpu.VMEM((CHUNK, D), jnp.float32),
    pltpu.VMEM_SHARED((NP, D), jnp.float32),
    pltpu.SemaphoreType.DMA((2,)),
    pltpu.SemaphoreType.DMA((2,)),
]

k_deg = pl.kernel(_deg_body, out_type=_DEG_OUT, mesh=_mesh,
                  scratch_types=_DEG_SCRATCH)
k_agg = pl.kernel(_agg_body, out_type=_AGG_OUT, mesh=_mesh,
                  scratch_types=_AGG_SCRATCH)


def _dis_block(degp):
    deg = 1.0 + degp[0, :, 0] + degp[1, :, 0]
    return lax.rsqrt(deg)[:, None]


def _xw_body(x_ref, w_ref, o_ref):
    o_ref[...] = jnp.dot(x_ref[...], w_ref[...],
                         preferred_element_type=jnp.float32,
                         precision=lax.Precision.HIGHEST)


def _scale_body(x_ref, degp_ref, o_ref):
    o_ref[...] = x_ref[...] * _dis_block(degp_ref[...])


def _mid_body(p_ref, y_ref, degp_ref, b_ref, w_ref, o_ref):
    dis = _dis_block(degp_ref[...])
    acc = p_ref[0] + p_ref[1] + y_ref[...]
    h = jnp.maximum(acc * dis + b_ref[...], 0.0)
    o_ref[...] = jnp.dot(h, w_ref[...],
                         preferred_element_type=jnp.float32,
                         precision=lax.Precision.HIGHEST) * dis


def _out_body(q_ref, y_ref, degp_ref, b_ref, o_ref):
    dis = _dis_block(degp_ref[...])
    o_ref[...] = (q_ref[0] + q_ref[1] + y_ref[...]) * dis + b_ref[...]


_row_spec = pl.BlockSpec((BLK, D), lambda i: (i, 0))
_pair_spec = pl.BlockSpec((2, BLK, D), lambda i: (0, i, 0))
_deg_spec = pl.BlockSpec((2, BLK, D), lambda i: (0, i, 0))
_w_spec = pl.BlockSpec((D, D), lambda i: (0, 0))
_b_spec = pl.BlockSpec((1, D), lambda i: (0, 0))
_GRID = (NP // BLK,)
_out128 = jax.ShapeDtypeStruct((NP, D), jnp.float32)

_k_xw = pl.pallas_call(
    _xw_body, grid=_GRID,
    in_specs=[_row_spec, _w_spec],
    out_specs=_row_spec, out_shape=_out128)

_k_scale = pl.pallas_call(
    _scale_body, grid=_GRID,
    in_specs=[_row_spec, _deg_spec],
    out_specs=_row_spec, out_shape=_out128)

_k_mid = pl.pallas_call(
    _mid_body, grid=_GRID,
    in_specs=[_pair_spec, _row_spec, _deg_spec, _b_spec, _w_spec],
    out_specs=_row_spec, out_shape=_out128)

_k_out = pl.pallas_call(
    _out_body, grid=_GRID,
    in_specs=[_pair_spec, _row_spec, _deg_spec, _b_spec],
    out_specs=_row_spec, out_shape=_out128)


def kernel(x, edge_index, W1, b1, W2, b2):
    ei = edge_index.astype(jnp.int32)
    pad = N + 1 + (jnp.arange(EP - E, dtype=jnp.int32) % (NP - N - 1))
    src = jnp.concatenate([ei[0], pad]).reshape(NW, CW, CHUNK)
    dst = jnp.concatenate([ei[1], pad]).reshape(NW, CW, CHUNK)
    eic = jnp.stack([src, dst], axis=2).reshape(NW * CW, 2, CHUNK)
    xp = jnp.pad(x, ((0, NP - N), (0, 0)))
    b1r = b1.reshape(1, D)
    b2r = b2.reshape(1, D)

    xw = _k_xw(xp, W1)
    degp = k_deg(dst)
    y1 = _k_scale(xw, degp)
    p1 = k_agg(y1, eic)
    y2 = _k_mid(p1, y1, degp, b1r, W2)
    p2 = k_agg(y2, eic)
    z = _k_out(p2, y2, degp, b2r)
    return z[:N]


# speedup vs baseline: 3.4986x; 1.0019x over previous
import functools

import jax
import jax.numpy as jnp
from jax import lax
from jax.experimental import pallas as pl
from jax.experimental.pallas import tpu as pltpu, tpu_sc as plsc

N = 10000
D = 128
E = 320000
NP = 10240
NW = 32
CHUNK = 128
CW = 80
EP = NW * CW * CHUNK
ROWS_PER_TILE = NP // 16
BLK = 5120

_mesh = plsc.VectorSubcoreMesh(core_axis_name="c", subcore_axis_name="s")


def _deg_body(dst_hbm, out_hbm, dst_v, ones_v, acc_sh, sem):
    cid = lax.axis_index("c")
    sid = lax.axis_index("s")
    wid = sid * 2 + cid
    pltpu.sync_copy(dst_hbm.at[wid], dst_v)

    lane = lax.iota(jnp.int32, 16)
    one16 = jnp.where(lane == 0, 1.0, 0.0).astype(jnp.float32)
    zero16 = jnp.zeros((16,), jnp.float32)

    def fill0(i, _):
        for l in range(D // 16):
            ones_v[i, pl.ds(l * 16, 16)] = zero16
        return 0
    lax.fori_loop(0, CHUNK, fill0, 0)

    def zb(b, _):
        pltpu.sync_copy(
            ones_v, acc_sh.at[pl.ds(sid * ROWS_PER_TILE + b * CHUNK, CHUNK)])
        return 0
    lax.fori_loop(0, ROWS_PER_TILE // CHUNK, zb, 0)

    def fill1(i, _):
        ones_v[i, pl.ds(0, 16)] = one16
        return 0
    lax.fori_loop(0, CHUNK, fill1, 0)
    plsc.subcore_barrier()

    pltpu.async_copy(ones_v, acc_sh.at[dst_v.at[0]], sem, add=True)

    def chunk(j, _):
        @pl.when(j + 1 < CW)
        def _():
            pltpu.async_copy(ones_v, acc_sh.at[dst_v.at[j + 1]], sem,
                             add=True)
        pltpu.make_async_copy(ones_v, acc_sh.at[dst_v.at[j]], sem).wait()
        return 0
    lax.fori_loop(0, CW, chunk, 0)
    plsc.subcore_barrier()

    r0 = sid * ROWS_PER_TILE
    pltpu.sync_copy(acc_sh.at[pl.ds(r0, ROWS_PER_TILE)],
                    out_hbm.at[cid].at[pl.ds(r0, ROWS_PER_TILE)])


def _agg_body(y_hbm, ei_hbm, out_hbm,
              ib0, ib1, rows_a, rows_b, acc_sh, sem_i, sem_g):
    cid = lax.axis_index("c")
    sid = lax.axis_index("s")
    wid = sid * 2 + cid
    base = wid * CW

    zero16 = jnp.zeros((16,), jnp.float32)

    def zr(i, _):
        for l in range(D // 16):
            rows_a[i, pl.ds(l * 16, 16)] = zero16
        return 0
    lax.fori_loop(0, CHUNK, zr, 0)

    def zb(b, _):
        pltpu.sync_copy(
            rows_a,
            acc_sh.at[pl.ds(sid * ROWS_PER_TILE + b * CHUNK, CHUNK)])
        return 0
    lax.fori_loop(0, ROWS_PER_TILE // CHUNK, zb, 0)
    plsc.subcore_barrier()

    ib = (ib0, ib1)
    rows = (rows_a, rows_b)

    pltpu.async_copy(ei_hbm.at[base + 0], ib0, sem_i.at[0])
    pltpu.async_copy(ei_hbm.at[base + 1], ib1, sem_i.at[1])
    pltpu.make_async_copy(ei_hbm.at[base], ib0, sem_i.at[0]).wait()
    pltpu.async_copy(y_hbm.at[ib0.at[0]], rows_a, sem_g.at[0])

    def chunk2(jj, _):
        for b in range(2):
            j = 2 * jj + b
            nb = 1 - b

            @pl.when(j + 1 < CW)
            def _():
                pltpu.make_async_copy(ei_hbm.at[base + j + 1], ib[nb],
                                      sem_i.at[nb]).wait()
                pltpu.async_copy(y_hbm.at[ib[nb].at[0]], rows[nb],
                                 sem_g.at[nb])

            pltpu.make_async_copy(y_hbm.at[ib[b].at[0]], rows[b],
                                  sem_g.at[b]).wait()
            pltpu.sync_copy(rows[b], acc_sh.at[ib[b].at[1]], add=True)

            @pl.when(j + 2 < CW)
            def _():
                pltpu.async_copy(ei_hbm.at[base + j + 2], ib[b],
                                 sem_i.at[b])
        return 0
    lax.fori_loop(0, CW // 2, chunk2, 0)
    plsc.subcore_barrier()

    r0 = sid * ROWS_PER_TILE
    pltpu.sync_copy(acc_sh.at[pl.ds(r0, ROWS_PER_TILE)],
                    out_hbm.at[cid].at[pl.ds(r0, ROWS_PER_TILE)])


_DEG_OUT = jax.ShapeDtypeStruct((2, NP, D), jnp.float32)
_DEG_SCRATCH = [
    pltpu.VMEM((CW, CHUNK), jnp.int32),
    pltpu.VMEM((CHUNK, D), jnp.float32),
    pltpu.VMEM_SHARED((NP, D), jnp.float32),
    pltpu.SemaphoreType.DMA,
]
_AGG_OUT = jax.ShapeDtypeStruct((2, NP, D), jnp.float32)
_AGG_SCRATCH = [
    pltpu.VMEM((2, CHUNK), jnp.int32),
    pltpu.VMEM((2, CHUNK), jnp.int32),
    pltpu.VMEM((CHUNK, D), jnp.float32),
    pltpu.VMEM((CHUNK, D), jnp.float32),
    pltpu.VMEM_SHARED((NP, D), jnp.float32),
    pltpu.SemaphoreType.DMA((2,)),
    pltpu.SemaphoreType.DMA((2,)),
]

k_deg = pl.kernel(_deg_body, out_type=_DEG_OUT, mesh=_mesh,
                  scratch_types=_DEG_SCRATCH)
k_agg = pl.kernel(_agg_body, out_type=_AGG_OUT, mesh=_mesh,
                  scratch_types=_AGG_SCRATCH)


def _dis_block(degp):
    deg = 1.0 + degp[0, :, 0] + degp[1, :, 0]
    return lax.rsqrt(deg)[:, None]


def _xw_body(x_ref, w_ref, o_ref):
    o_ref[...] = jnp.dot(x_ref[...], w_ref[...],
                         preferred_element_type=jnp.float32,
                         precision=lax.Precision.HIGHEST)


def _scale_body(x_ref, degp_ref, o_ref):
    o_ref[...] = x_ref[...] * _dis_block(degp_ref[...])


def _mid_body(p_ref, y_ref, degp_ref, b_ref, w_ref, o_ref):
    dis = _dis_block(degp_ref[...])
    acc = p_ref[0] + p_ref[1] + y_ref[...]
    h = jnp.maximum(acc * dis + b_ref[...], 0.0)
    o_ref[...] = jnp.dot(h, w_ref[...],
                         preferred_element_type=jnp.float32,
                         precision=lax.Precision.HIGHEST) * dis


def _out_body(q_ref, y_ref, degp_ref, b_ref, o_ref):
    dis = _dis_block(degp_ref[...])
    o_ref[...] = (q_ref[0] + q_ref[1] + y_ref[...]) * dis + b_ref[...]


_row_spec = pl.BlockSpec((BLK, D), lambda i: (i, 0))
_pair_spec = pl.BlockSpec((2, BLK, D), lambda i: (0, i, 0))
_deg_spec = pl.BlockSpec((2, BLK, D), lambda i: (0, i, 0))
_w_spec = pl.BlockSpec((D, D), lambda i: (0, 0))
_b_spec = pl.BlockSpec((1, D), lambda i: (0, 0))
_GRID = (NP // BLK,)
_out128 = jax.ShapeDtypeStruct((NP, D), jnp.float32)

_k_xw = pl.pallas_call(
    _xw_body, grid=_GRID,
    in_specs=[_row_spec, _w_spec],
    out_specs=_row_spec, out_shape=_out128)

_k_scale = pl.pallas_call(
    _scale_body, grid=_GRID,
    in_specs=[_row_spec, _deg_spec],
    out_specs=_row_spec, out_shape=_out128)

_k_mid = pl.pallas_call(
    _mid_body, grid=_GRID,
    in_specs=[_pair_spec, _row_spec, _deg_spec, _b_spec, _w_spec],
    out_specs=_row_spec, out_shape=_out128)

_k_out = pl.pallas_call(
    _out_body, grid=_GRID,
    in_specs=[_pair_spec, _row_spec, _deg_spec, _b_spec],
    out_specs=_row_spec, out_shape=_out128)


def kernel(x, edge_index, W1, b1, W2, b2):
    ei = edge_index.astype(jnp.int32)
    pad = N + 1 + (jnp.arange(EP - E, dtype=jnp.int32) % (NP - N - 1))
    src = jnp.concatenate([ei[0], pad]).reshape(NW, CW, CHUNK)
    dst = jnp.concatenate([ei[1], pad]).reshape(NW, CW, CHUNK)
    eic = jnp.stack([src, dst], axis=2).reshape(NW * CW, 2, CHUNK)
    xp = jnp.pad(x, ((0, NP - N), (0, 0)))
    b1r = b1.reshape(1, D)
    b2r = b2.reshape(1, D)

    xw = _k_xw(xp, W1)
    degp = k_deg(dst)
    y1 = _k_scale(xw, degp)
    p1 = k_agg(y1, eic)
    y2 = _k_mid(p1, y1, degp, b1r, W2)
    p2 = k_agg(y2, eic)
    z = _k_out(p2, y2, degp, b2r)
    return z[:N]
